# baseline (speedup 1.0000x reference)
import jax
import jax.numpy as jnp
from jax import lax
from jax.experimental import pallas as pl
from jax.experimental.pallas import tpu as pltpu
from jax.experimental.pallas import tpu_sc as plsc

_N, _L, _V = 8, 512, 32000
_B = _N * _L
_NC = 2
_NS = 16
_NW = _NC * _NS
_CHUNK = _B // _NW
_LANES = 16
_NVEC = _CHUNK // _LANES


def _partials_body(lp_hbm, tgt_hbm, msk_hbm, out_hbm,
                   idx_v, msk_v, val_v, part_v, sem):
    wid = lax.axis_index("s") * _NC + lax.axis_index("c")
    base = wid * _CHUNK

    pltpu.sync_copy(tgt_hbm.at[pl.ds(base, _CHUNK)], idx_v)
    pltpu.sync_copy(msk_hbm.at[pl.ds(base, _CHUNK)], msk_v)

    for j in range(_NVEC):
        sl = pl.ds(j * _LANES, _LANES)
        pos = base + j * _LANES + lax.iota(jnp.int32, _LANES)
        idx_v[sl] = pos * _V + idx_v[sl]

    pltpu.async_copy(lp_hbm.at[idx_v], val_v, sem).wait()

    accw = jnp.zeros((_LANES,), jnp.float32)
    accm = jnp.zeros((_LANES,), jnp.float32)
    for j in range(_NVEC):
        sl = pl.ds(j * _LANES, _LANES)
        accw = accw + val_v[sl] * msk_v[sl]
        accm = accm + msk_v[sl]
    part_v[0, :] = accw
    part_v[1, :] = accm
    pltpu.sync_copy(part_v, out_hbm.at[wid])


def _finalize_body(parts_ref, out_ref):
    ws = jnp.sum(parts_ref[:, 0, :])
    ms = jnp.sum(parts_ref[:, 1, :])
    out_ref[...] = jnp.full((1, 1), -ws / (ms + 1e-6), jnp.float32)


def kernel(logprobs, target, mask):
    lp = logprobs.reshape(_B * _V)
    tgt = target.astype(jnp.int32).reshape(_B)
    msk = mask.astype(jnp.float32).reshape(_B)

    mesh = plsc.VectorSubcoreMesh(core_axis_name="c", subcore_axis_name="s")
    parts = pl.kernel(
        _partials_body,
        out_type=jax.ShapeDtypeStruct((_NW, 2, _LANES), jnp.float32),
        mesh=mesh,
        scratch_types=[
            pltpu.VMEM((_CHUNK,), jnp.int32),
            pltpu.VMEM((_CHUNK,), jnp.float32),
            pltpu.VMEM((_CHUNK,), jnp.float32),
            pltpu.VMEM((2, _LANES), jnp.float32),
            pltpu.SemaphoreType.DMA,
        ],
    )(lp, tgt, msk)

    out = pl.pallas_call(
        _finalize_body,
        out_shape=jax.ShapeDtypeStruct((1, 1), jnp.float32),
    )(parts)
    return out[0, 0]


# speedup vs baseline: 11.9668x; 11.9668x over previous
import jax
import jax.numpy as jnp
from jax import lax
from jax.experimental import pallas as pl
from jax.experimental.pallas import tpu as pltpu
from jax.experimental.pallas import tpu_sc as plsc

_N, _L, _V = 8, 512, 32000
_B = _N * _L
_NC = 2
_NS = 16
_NW = _NC * _NS
_CHUNK = _B // _NW
_LANES = 16
_TILE_R, _TILE_C = 8, 128
_ROUND = 64
_NROUND = _CHUNK // _ROUND


def _partials_body(lp_hbm, tgt_hbm, msk_hbm, out_hbm,
                   idx_v, msk_v, val_v, gran_v, part_v, sem):
    wid = lax.axis_index("s") * _NC + lax.axis_index("c")
    base = wid * _CHUNK

    pltpu.sync_copy(tgt_hbm.at[pl.ds(base, _CHUNK)], idx_v)
    pltpu.sync_copy(msk_hbm.at[pl.ds(base, _CHUNK)], msk_v)

    lane = lax.iota(jnp.int32, _LANES)
    for rnd in range(_NROUND):
        copies = []
        for j in range(_ROUND // _LANES):
            gbase = rnd * _ROUND + j * _LANES
            tv = idx_v[pl.ds(gbase, _LANES)]
            ctv = lax.shift_right_logical(tv, 7)
            for i in range(_LANES):
                p = j * _LANES + i
                row0 = pl.multiple_of(base + (gbase + i) - ((gbase + i) % _TILE_R),
                                      _TILE_R)
                col0 = pl.multiple_of(ctv[i] * _TILE_C, _TILE_C)
                copies.append(pltpu.async_copy(
                    lp_hbm.at[pl.ds(row0, _TILE_R), pl.ds(col0, _TILE_C)],
                    gran_v.at[p], sem))
        for c in copies:
            c.wait()
        for j in range(_ROUND // _LANES):
            gbase = rnd * _ROUND + j * _LANES
            tv = idx_v[pl.ds(gbase, _LANES)]
            cols = tv & (_TILE_C - 1)
            vacc = jnp.zeros((_LANES,), jnp.float32)
            for i in range(_LANES):
                p = j * _LANES + i
                start = cols[i] - i
                v16 = gran_v[p, i & (_TILE_R - 1), pl.ds(start, _LANES)]
                vacc = jnp.where(lane == i, v16, vacc)
            val_v[pl.ds(gbase, _LANES)] = vacc

    accw = jnp.zeros((_LANES,), jnp.float32)
    accm = jnp.zeros((_LANES,), jnp.float32)
    for j in range(_CHUNK // _LANES):
        sl = pl.ds(j * _LANES, _LANES)
        accw = accw + val_v[sl] * msk_v[sl]
        accm = accm + msk_v[sl]
    part_v[0, :] = accw
    part_v[1, :] = accm
    pltpu.sync_copy(part_v, out_hbm.at[wid])


def _finalize_body(parts_ref, out_ref):
    ws = jnp.sum(parts_ref[:, 0, :])
    ms = jnp.sum(parts_ref[:, 1, :])
    out_ref[...] = jnp.full((1, 1), -ws / (ms + 1e-6), jnp.float32)


def kernel(logprobs, target, mask):
    lp = logprobs.reshape(_B, _V)
    tgt = target.astype(jnp.int32).reshape(_B)
    msk = mask.astype(jnp.float32).reshape(_B)

    mesh = plsc.VectorSubcoreMesh(core_axis_name="c", subcore_axis_name="s")
    parts = pl.kernel(
        _partials_body,
        out_type=jax.ShapeDtypeStruct((_NW, 2, _LANES), jnp.float32),
        mesh=mesh,
        scratch_types=[
            pltpu.VMEM((_CHUNK,), jnp.int32),
            pltpu.VMEM((_CHUNK,), jnp.float32),
            pltpu.VMEM((_CHUNK,), jnp.float32),
            pltpu.VMEM((_ROUND + 1, _TILE_R, _TILE_C), jnp.float32),
            pltpu.VMEM((2, _LANES), jnp.float32),
            pltpu.SemaphoreType.DMA,
        ],
    )(lp, tgt, msk)

    out = pl.pallas_call(
        _finalize_body,
        out_shape=jax.ShapeDtypeStruct((1, 1), jnp.float32),
    )(parts)
    return out[0, 0]
